# baseline (speedup 1.0000x reference)
import functools

import jax
import jax.numpy as jnp
from jax import lax
from jax.experimental import pallas as pl
from jax.experimental.pallas import tpu as pltpu
from jax.experimental.pallas import tpu_sc as plsc

T = 2048
D = 2048
NH = 16
HD = 128
NA = 1024
REFF = 512
NKEEP = T - REFF
BN = 256
NW = 32
ROWS_PER = NKEEP // NW
UROWS = T // NW

_SCALE = HD ** -0.5

@functools.lru_cache(maxsize=None)
def _sc_mesh():
    return plsc.VectorSubcoreMesh(core_axis_name="c", subcore_axis_name="s")


def _m1_body(a_ref, b_ref, bb_ref, rank_ref, cume_ref, bs_ref):
    j = pl.program_id(0)

    @pl.when(j == 0)
    def _():
        bs_ref[...] = jnp.full((NA, 1), -jnp.inf, jnp.float32)
        bb_ref[...] = jnp.zeros((NA, 1), jnp.int32)

    sim = lax.dot_general(
        a_ref[...], b_ref[...], (((1,), (1,)), ((), ())),
        preferred_element_type=jnp.float32,
    )
    bmax = jnp.max(sim, axis=1, keepdims=True)
    ids = lax.broadcasted_iota(jnp.int32, (NA, BN), 1) + j * BN
    bloc = jnp.min(jnp.where(sim == bmax, ids, NA), axis=1, keepdims=True)
    upd = bmax > bs_ref[...]
    bb_ref[...] = jnp.where(upd, bloc, bb_ref[...])
    bs_ref[...] = jnp.maximum(bmax, bs_ref[...])

    @pl.when(j == NA // BN - 1)
    def _():
        bs = bs_ref[...]
        rid = lax.broadcasted_iota(jnp.int32, (NA, NA), 0)
        cid = lax.broadcasted_iota(jnp.int32, (NA, NA), 1)
        ident = (rid == cid).astype(jnp.float32)
        nn = (((1,), (0,)), ((), ()))
        bs_row = lax.dot_general(
            jnp.ones((1, NA), jnp.float32), ident * bs, nn,
            preferred_element_type=jnp.float32,
            precision=lax.Precision.HIGHEST,
        )
        b_j = jnp.broadcast_to(bs_row, (NA, NA))
        b_i = jnp.broadcast_to(bs, (NA, NA))
        grt = (b_j > b_i) | ((b_j == b_i) & (cid < rid))
        rank = jnp.sum(grt.astype(jnp.float32), axis=1, keepdims=True)
        keep_e = jnp.where(rank >= float(REFF), 1.0, 0.0)
        tri = (cid <= rid).astype(jnp.float32)
        cume = lax.dot_general(
            tri, keep_e, nn,
            preferred_element_type=jnp.float32,
            precision=lax.Precision.HIGHEST,
        )
        rank_ref[...] = rank.astype(jnp.int32)
        cume_ref[...] = cume.astype(jnp.int32)


def _match_call(a_x, b_x):
    return pl.pallas_call(
        _m1_body,
        grid=(NA // BN,),
        in_specs=[
            pl.BlockSpec((NA, D), lambda j: (0, 0)),
            pl.BlockSpec((BN, D), lambda j: (j, 0)),
        ],
        out_specs=[
            pl.BlockSpec((NA, 1), lambda j: (0, 0)),
            pl.BlockSpec((NA, 1), lambda j: (0, 0)),
            pl.BlockSpec((NA, 1), lambda j: (0, 0)),
        ],
        out_shape=[
            jax.ShapeDtypeStruct((NA, 1), jnp.int32),
            jax.ShapeDtypeStruct((NA, 1), jnp.int32),
            jax.ShapeDtypeStruct((NA, 1), jnp.int32),
        ],
        scratch_shapes=[pltpu.VMEM((NA, 1), jnp.float32)],
    )(a_x, b_x)


def _sc_route_body(x_hbm, rank_hbm, bb_hbm, cume_hbm, xm_hbm, lookup_hbm,
                   rank_v, bb_v, cume_v, src_v, dst_v, kidx_v, lk_v, rows_v,
                   sem):
    wid = lax.axis_index("s") * 2 + lax.axis_index("c")
    lo = wid * ROWS_PER

    pltpu.sync_copy(rank_hbm, rank_v)
    pltpu.sync_copy(bb_hbm, bb_v)
    pltpu.sync_copy(cume_hbm, cume_v)

    def body1(it, carry):
        i16 = jnp.arange(16, dtype=jnp.int32) + it * 16
        r16 = rank_v[pl.ds(it * 16, 16)]
        b16 = bb_v[pl.ds(it * 16, 16)]
        m = r16 < REFF
        plsc.store_scatter(src_v, [r16], 2 * i16, mask=m)
        plsc.store_scatter(dst_v, [r16], 2 * b16 + 1, mask=m)
        return carry

    lax.fori_loop(0, NA // 16, body1, 0)

    def body2(it, carry):
        t16 = jnp.arange(16, dtype=jnp.int32) + it * 16
        i16 = t16 >> 1
        odd = t16 & 1
        rk = plsc.load_gather(rank_v, [i16])
        ce = plsc.load_gather(cume_v, [i16])
        keep = (odd == 1) | (rk >= REFF)
        c = i16 + ce - 1 + odd
        plsc.store_scatter(kidx_v, [c], t16, mask=keep)
        lk_v[pl.ds(it * 16, 16)] = c
        return carry

    lax.fori_loop(0, T // 16, body2, 0)

    def body3(it, carry):
        s16 = src_v[pl.ds(it * 16, 16)]
        d16 = dst_v[pl.ds(it * 16, 16)]
        di = d16 >> 1
        ce = plsc.load_gather(cume_v, [di])
        plsc.store_scatter(lk_v, [s16], di + ce)
        return carry

    lax.fori_loop(0, REFF // 16, body3, 0)

    pltpu.async_copy(x_hbm.at[kidx_v.at[pl.ds(lo, ROWS_PER)]], rows_v, sem).wait()

    def body4(it, carry):
        r16 = jnp.arange(16, dtype=jnp.int32) + it * 16
        i16 = r16 >> 1
        odd = r16 & 1
        rk = plsc.load_gather(rank_v, [i16])
        ce = plsc.load_gather(cume_v, [i16])
        keep = (odd == 1) | (rk >= REFF)
        c = i16 + ce - 1 + odd
        m = keep & (c >= lo) & (c < lo + ROWS_PER)
        loc = c - lo
        s16 = src_v[pl.ds(it * 16, 16)]
        d16 = dst_v[pl.ds(it * 16, 16)]
        xs = plsc.load_gather(rows_v, [loc, s16], mask=m)
        xd = plsc.load_gather(rows_v, [loc, d16], mask=m)
        plsc.store_scatter(rows_v, [loc, d16], (xs + xd) * 0.5, mask=m)
        return carry

    lax.fori_loop(0, REFF // 16, body4, 0)

    pltpu.sync_copy(rows_v, xm_hbm.at[pl.ds(lo, ROWS_PER)])

    @pl.when(wid == 0)
    def _():
        pltpu.sync_copy(lk_v, lookup_hbm)


@functools.lru_cache(maxsize=None)
def _sc_route():
    return pl.kernel(
        _sc_route_body,
        out_type=(
            jax.ShapeDtypeStruct((NKEEP, D), jnp.float32),
            jax.ShapeDtypeStruct((T,), jnp.int32),
        ),
        mesh=_sc_mesh(),
        compiler_params=pltpu.CompilerParams(needs_layout_passes=False),
        scratch_types=[
            pltpu.VMEM((NA,), jnp.int32),
            pltpu.VMEM((NA,), jnp.int32),
            pltpu.VMEM((NA,), jnp.int32),
            pltpu.VMEM((REFF,), jnp.int32),
            pltpu.VMEM((REFF,), jnp.int32),
            pltpu.VMEM((NKEEP,), jnp.int32),
            pltpu.VMEM((T,), jnp.int32),
            pltpu.VMEM((ROWS_PER, D), jnp.float32),
            pltpu.SemaphoreType.DMA,
        ],
    )


def _sc_route_call(x2, rank, bb, cume):
    return _sc_route()(x2, rank, bb, cume)


def _fused_body(xm_ref, wq_ref, wk_ref, wv_ref, wo_ref, out_ref):
    h = pl.program_id(0)
    nt = (((1,), (1,)), ((), ()))
    nn = (((1,), (0,)), ((), ()))
    xm = xm_ref[...]
    q = lax.dot_general(xm, wq_ref[...].astype(jnp.bfloat16), nt,
                        preferred_element_type=jnp.float32).astype(jnp.bfloat16)
    k = lax.dot_general(xm, wk_ref[...].astype(jnp.bfloat16), nt,
                        preferred_element_type=jnp.float32).astype(jnp.bfloat16)
    v = lax.dot_general(xm, wv_ref[...].astype(jnp.bfloat16), nt,
                        preferred_element_type=jnp.float32).astype(jnp.bfloat16)
    logits = lax.dot_general(q, k, nt, preferred_element_type=jnp.float32)
    p = jnp.exp(logits * _SCALE)
    s = jnp.sum(p, axis=1, keepdims=True)
    o = lax.dot_general(p.astype(jnp.bfloat16), v, nn,
                        preferred_element_type=jnp.float32)
    o = (o / s).astype(jnp.bfloat16)
    proj = lax.dot_general(o, wo_ref[...].astype(jnp.bfloat16), nt,
                           preferred_element_type=jnp.float32)

    @pl.when(h == 0)
    def _():
        out_ref[...] = proj

    @pl.when(h > 0)
    def _():
        out_ref[...] += proj


def _fused_call(xm_bf, w_qkv, w_out):
    return pl.pallas_call(
        _fused_body,
        grid=(NH,),
        in_specs=[
            pl.BlockSpec((NKEEP, D), lambda h: (0, 0)),
            pl.BlockSpec((HD, D), lambda h: (h, 0)),
            pl.BlockSpec((HD, D), lambda h: (NH + h, 0)),
            pl.BlockSpec((HD, D), lambda h: (2 * NH + h, 0)),
            pl.BlockSpec((D, HD), lambda h: (0, h)),
        ],
        out_specs=pl.BlockSpec((NKEEP, D), lambda h: (0, 0)),
        out_shape=jax.ShapeDtypeStruct((NKEEP, D), jnp.float32),
    )(xm_bf, w_qkv, w_qkv, w_qkv, w_out)


def _sc_unmerge_body(outr_hbm, lookup_hbm, out_hbm, lk_v, buf0, buf1,
                     sem0, sem1):
    wid = lax.axis_index("s") * 2 + lax.axis_index("c")
    lo = wid * UROWS
    pltpu.sync_copy(lookup_hbm.at[pl.ds(lo, UROWS)], lk_v)

    def start(ch, buf, sem):
        return pltpu.async_copy(outr_hbm.at[lk_v.at[pl.ds(ch * 16, 16)]], buf, sem)

    def drain(cp, ch, buf):
        cp.wait()
        pltpu.sync_copy(buf, out_hbm.at[pl.ds(lo + ch * 16, 16)])

    c0 = start(0, buf0, sem0)
    c1 = start(1, buf1, sem1)
    drain(c0, 0, buf0)
    c2 = start(2, buf0, sem0)
    drain(c1, 1, buf1)
    c3 = start(3, buf1, sem1)
    drain(c2, 2, buf0)
    drain(c3, 3, buf1)


@functools.lru_cache(maxsize=None)
def _sc_unmerge():
    return pl.kernel(
        _sc_unmerge_body,
        out_type=jax.ShapeDtypeStruct((T, D), jnp.float32),
        mesh=_sc_mesh(),
        compiler_params=pltpu.CompilerParams(needs_layout_passes=False),
        scratch_types=[
            pltpu.VMEM((UROWS,), jnp.int32),
            pltpu.VMEM((16, D), jnp.float32),
            pltpu.VMEM((16, D), jnp.float32),
            pltpu.SemaphoreType.DMA,
            pltpu.SemaphoreType.DMA,
        ],
    )


def _sc_unmerge_call(outr, lookup):
    return _sc_unmerge()(outr, lookup)


def kernel(x, W_qkv, W_out):
    x2 = x[0]
    a_x = x2[0::2]
    b_x = x2[1::2]
    an = a_x / jnp.maximum(jnp.linalg.norm(a_x, axis=-1, keepdims=True), 1e-12)
    bn = b_x / jnp.maximum(jnp.linalg.norm(b_x, axis=-1, keepdims=True), 1e-12)
    bb, rank, cume = _match_call(an.astype(jnp.bfloat16), bn.astype(jnp.bfloat16))
    xm, lookup = _sc_route_call(
        x2, rank.reshape(NA), bb.reshape(NA), cume.reshape(NA))
    outr = _fused_call(xm.astype(jnp.bfloat16), W_qkv, W_out)
    out = _sc_unmerge_call(outr, lookup)
    return out.reshape(1, T, D)


# speedup vs baseline: 1.0839x; 1.0839x over previous
import functools

import jax
import jax.numpy as jnp
from jax import lax
from jax.experimental import pallas as pl
from jax.experimental.pallas import tpu as pltpu
from jax.experimental.pallas import tpu_sc as plsc

T = 2048
D = 2048
NH = 16
HD = 128
NA = 1024
REFF = 512
NKEEP = T - REFF
BN = 256
NW = 32
ROWS_PER = NKEEP // NW
UROWS = T // NW

_SCALE = HD ** -0.5

@functools.lru_cache(maxsize=None)
def _sc_mesh():
    return plsc.VectorSubcoreMesh(core_axis_name="c", subcore_axis_name="s")


def _m1_body(a_ref, b_ref, bb_ref, rank_ref, cume_ref, bs_ref):
    j = pl.program_id(0)

    @pl.when(j == 0)
    def _():
        bs_ref[...] = jnp.full((NA, 1), -jnp.inf, jnp.float32)
        bb_ref[...] = jnp.zeros((NA, 1), jnp.int32)

    sim = lax.dot_general(
        a_ref[...], b_ref[...], (((1,), (1,)), ((), ())),
        preferred_element_type=jnp.float32,
    )
    bmax = jnp.max(sim, axis=1, keepdims=True)
    ids = lax.broadcasted_iota(jnp.int32, (NA, BN), 1) + j * BN
    bloc = jnp.min(jnp.where(sim == bmax, ids, NA), axis=1, keepdims=True)
    upd = bmax > bs_ref[...]
    bb_ref[...] = jnp.where(upd, bloc, bb_ref[...])
    bs_ref[...] = jnp.maximum(bmax, bs_ref[...])

    @pl.when(j == NA // BN - 1)
    def _():
        bs = bs_ref[...]
        rid = lax.broadcasted_iota(jnp.int32, (NA, NA), 0)
        cid = lax.broadcasted_iota(jnp.int32, (NA, NA), 1)
        ident = (rid == cid).astype(jnp.float32)
        nn = (((1,), (0,)), ((), ()))
        bs_row = lax.dot_general(
            jnp.ones((1, NA), jnp.float32), ident * bs, nn,
            preferred_element_type=jnp.float32,
            precision=lax.Precision.HIGHEST,
        )
        b_j = jnp.broadcast_to(bs_row, (NA, NA))
        b_i = jnp.broadcast_to(bs, (NA, NA))
        grt = (b_j > b_i) | ((b_j == b_i) & (cid < rid))
        rank = jnp.sum(grt.astype(jnp.float32), axis=1, keepdims=True)
        keep_e = jnp.where(rank >= float(REFF), 1.0, 0.0)
        tri = (cid <= rid).astype(jnp.float32)
        cume = lax.dot_general(
            tri, keep_e, nn,
            preferred_element_type=jnp.float32,
            precision=lax.Precision.HIGHEST,
        )
        rank_ref[...] = rank.astype(jnp.int32)
        cume_ref[...] = cume.astype(jnp.int32)


def _match_call(a_x, b_x):
    return pl.pallas_call(
        _m1_body,
        grid=(NA // BN,),
        in_specs=[
            pl.BlockSpec((NA, D), lambda j: (0, 0)),
            pl.BlockSpec((BN, D), lambda j: (j, 0)),
        ],
        out_specs=[
            pl.BlockSpec((NA, 1), lambda j: (0, 0)),
            pl.BlockSpec((NA, 1), lambda j: (0, 0)),
            pl.BlockSpec((NA, 1), lambda j: (0, 0)),
        ],
        out_shape=[
            jax.ShapeDtypeStruct((NA, 1), jnp.int32),
            jax.ShapeDtypeStruct((NA, 1), jnp.int32),
            jax.ShapeDtypeStruct((NA, 1), jnp.int32),
        ],
        scratch_shapes=[pltpu.VMEM((NA, 1), jnp.float32)],
    )(a_x, b_x)


def _sc_route_body(x_hbm, rank_hbm, bb_hbm, cume_hbm, xm_hbm, lookup_hbm,
                   rank_v, bb_v, cume_v, src_v, dst_v, kidx_v, lk_v, rows_v,
                   sem):
    wid = lax.axis_index("s") * 2 + lax.axis_index("c")
    lo = wid * ROWS_PER

    pltpu.sync_copy(rank_hbm, rank_v)
    pltpu.sync_copy(bb_hbm, bb_v)
    pltpu.sync_copy(cume_hbm, cume_v)

    def body1(it, carry):
        i16 = jnp.arange(16, dtype=jnp.int32) + it * 16
        r16 = rank_v[pl.ds(it * 16, 16)]
        b16 = bb_v[pl.ds(it * 16, 16)]
        m = r16 < REFF
        plsc.store_scatter(src_v, [r16], 2 * i16, mask=m)
        plsc.store_scatter(dst_v, [r16], 2 * b16 + 1, mask=m)
        return carry

    lax.fori_loop(0, NA // 16, body1, 0)

    def body2(it, carry):
        t16 = jnp.arange(16, dtype=jnp.int32) + it * 16
        i16 = t16 >> 1
        odd = t16 & 1
        rk = plsc.load_gather(rank_v, [i16])
        ce = plsc.load_gather(cume_v, [i16])
        keep = (odd == 1) | (rk >= REFF)
        c = i16 + ce - 1 + odd
        plsc.store_scatter(kidx_v, [c], t16, mask=keep)
        lk_v[pl.ds(it * 16, 16)] = c
        return carry

    lax.fori_loop(0, T // 16, body2, 0)

    def body3(it, carry):
        s16 = src_v[pl.ds(it * 16, 16)]
        d16 = dst_v[pl.ds(it * 16, 16)]
        di = d16 >> 1
        ce = plsc.load_gather(cume_v, [di])
        plsc.store_scatter(lk_v, [s16], di + ce)
        return carry

    lax.fori_loop(0, REFF // 16, body3, 0)

    pltpu.async_copy(x_hbm.at[kidx_v.at[pl.ds(lo, ROWS_PER)]], rows_v, sem).wait()

    def body4(it, carry):
        r16 = jnp.arange(16, dtype=jnp.int32) + it * 16
        i16 = r16 >> 1
        odd = r16 & 1
        rk = plsc.load_gather(rank_v, [i16])
        ce = plsc.load_gather(cume_v, [i16])
        keep = (odd == 1) | (rk >= REFF)
        c = i16 + ce - 1 + odd
        m = keep & (c >= lo) & (c < lo + ROWS_PER)
        loc = c - lo
        s16 = src_v[pl.ds(it * 16, 16)]
        d16 = dst_v[pl.ds(it * 16, 16)]
        xs = plsc.load_gather(rows_v, [loc, s16], mask=m)
        xd = plsc.load_gather(rows_v, [loc, d16], mask=m)
        plsc.store_scatter(rows_v, [loc, d16], (xs + xd) * 0.5, mask=m)
        return carry

    lax.fori_loop(0, REFF // 16, body4, 0)

    pltpu.sync_copy(rows_v, xm_hbm.at[pl.ds(lo, ROWS_PER)])

    @pl.when(wid == 0)
    def _():
        pltpu.sync_copy(lk_v, lookup_hbm)


@functools.lru_cache(maxsize=None)
def _sc_route():
    return pl.kernel(
        _sc_route_body,
        out_type=(
            jax.ShapeDtypeStruct((NKEEP, D), jnp.float32),
            jax.ShapeDtypeStruct((T,), jnp.int32),
        ),
        mesh=_sc_mesh(),
        compiler_params=pltpu.CompilerParams(needs_layout_passes=False),
        scratch_types=[
            pltpu.VMEM((NA,), jnp.int32),
            pltpu.VMEM((NA,), jnp.int32),
            pltpu.VMEM((NA,), jnp.int32),
            pltpu.VMEM((REFF,), jnp.int32),
            pltpu.VMEM((REFF,), jnp.int32),
            pltpu.VMEM((NKEEP,), jnp.int32),
            pltpu.VMEM((T,), jnp.int32),
            pltpu.VMEM((ROWS_PER, D), jnp.float32),
            pltpu.SemaphoreType.DMA,
        ],
    )


def _sc_route_call(x2, rank, bb, cume):
    return _sc_route()(x2, rank, bb, cume)


def _qkv_body(xm_ref, w_ref, out_ref, xbf_ref):
    @pl.when(pl.program_id(0) == 0)
    def _():
        xbf_ref[...] = xm_ref[...].astype(jnp.bfloat16)

    w = w_ref[...].astype(jnp.bfloat16)
    out_ref[...] = lax.dot_general(
        xbf_ref[...], w, (((1,), (1,)), ((), ())),
        preferred_element_type=jnp.float32,
    ).astype(jnp.bfloat16)


def _qkv_call(xm, w_qkv):
    nblk = (3 * D) // 512
    return pl.pallas_call(
        _qkv_body,
        grid=(nblk,),
        in_specs=[
            pl.BlockSpec((NKEEP, D), lambda j: (0, 0)),
            pl.BlockSpec((512, D), lambda j: (j, 0)),
        ],
        out_specs=pl.BlockSpec((NKEEP, 512), lambda j: (0, j)),
        out_shape=jax.ShapeDtypeStruct((NKEEP, 3 * D), jnp.bfloat16),
        scratch_shapes=[pltpu.VMEM((NKEEP, D), jnp.bfloat16)],
    )(xm, w_qkv)


def _fused_body(q_ref, k_ref, v_ref, wo_ref, out_ref):
    h = pl.program_id(0)
    nt = (((1,), (1,)), ((), ()))
    nn = (((1,), (0,)), ((), ()))
    q = q_ref[...]
    k = k_ref[...]
    v = v_ref[...]
    logits = lax.dot_general(q, k, nt, preferred_element_type=jnp.float32)
    p = jnp.exp(logits * _SCALE)
    s = jnp.sum(p, axis=1, keepdims=True)
    o = lax.dot_general(p.astype(jnp.bfloat16), v, nn,
                        preferred_element_type=jnp.float32)
    o = (o / s).astype(jnp.bfloat16)
    proj = lax.dot_general(o, wo_ref[...].astype(jnp.bfloat16), nt,
                           preferred_element_type=jnp.float32)

    @pl.when(h == 0)
    def _():
        out_ref[...] = proj

    @pl.when(h > 0)
    def _():
        out_ref[...] += proj


def _fused_call(qkv, w_out):
    return pl.pallas_call(
        _fused_body,
        grid=(NH,),
        in_specs=[
            pl.BlockSpec((NKEEP, HD), lambda h: (0, h)),
            pl.BlockSpec((NKEEP, HD), lambda h: (0, NH + h)),
            pl.BlockSpec((NKEEP, HD), lambda h: (0, 2 * NH + h)),
            pl.BlockSpec((D, HD), lambda h: (0, h)),
        ],
        out_specs=pl.BlockSpec((NKEEP, D), lambda h: (0, 0)),
        out_shape=jax.ShapeDtypeStruct((NKEEP, D), jnp.float32),
    )(qkv, qkv, qkv, w_out)


def _sc_unmerge_body(outr_hbm, lookup_hbm, out_hbm, lk_v, buf0, buf1,
                     sem0, sem1):
    wid = lax.axis_index("s") * 2 + lax.axis_index("c")
    lo = wid * UROWS
    pltpu.sync_copy(lookup_hbm.at[pl.ds(lo, UROWS)], lk_v)

    def start(ch, buf, sem):
        return pltpu.async_copy(outr_hbm.at[lk_v.at[pl.ds(ch * 16, 16)]], buf, sem)

    def drain(cp, ch, buf):
        cp.wait()
        pltpu.sync_copy(buf, out_hbm.at[pl.ds(lo + ch * 16, 16)])

    c0 = start(0, buf0, sem0)
    c1 = start(1, buf1, sem1)
    drain(c0, 0, buf0)
    c2 = start(2, buf0, sem0)
    drain(c1, 1, buf1)
    c3 = start(3, buf1, sem1)
    drain(c2, 2, buf0)
    drain(c3, 3, buf1)


@functools.lru_cache(maxsize=None)
def _sc_unmerge():
    return pl.kernel(
        _sc_unmerge_body,
        out_type=jax.ShapeDtypeStruct((T, D), jnp.float32),
        mesh=_sc_mesh(),
        compiler_params=pltpu.CompilerParams(needs_layout_passes=False),
        scratch_types=[
            pltpu.VMEM((UROWS,), jnp.int32),
            pltpu.VMEM((16, D), jnp.float32),
            pltpu.VMEM((16, D), jnp.float32),
            pltpu.SemaphoreType.DMA,
            pltpu.SemaphoreType.DMA,
        ],
    )


def _sc_unmerge_call(outr, lookup):
    return _sc_unmerge()(outr, lookup)


def kernel(x, W_qkv, W_out):
    x2 = x[0]
    a_x = x2[0::2]
    b_x = x2[1::2]
    an = a_x / jnp.maximum(jnp.linalg.norm(a_x, axis=-1, keepdims=True), 1e-12)
    bn = b_x / jnp.maximum(jnp.linalg.norm(b_x, axis=-1, keepdims=True), 1e-12)
    bb, rank, cume = _match_call(an.astype(jnp.bfloat16), bn.astype(jnp.bfloat16))
    xm, lookup = _sc_route_call(
        x2, rank.reshape(NA), bb.reshape(NA), cume.reshape(NA))
    qkv = _qkv_call(xm, W_qkv)
    outr = _fused_call(qkv, W_out)
    out = _sc_unmerge_call(outr, lookup)
    return out.reshape(1, T, D)


# speedup vs baseline: 1.6777x; 1.5478x over previous
import functools

import jax
import jax.numpy as jnp
from jax import lax
from jax.experimental import pallas as pl
from jax.experimental.pallas import tpu as pltpu
from jax.experimental.pallas import tpu_sc as plsc

T = 2048
D = 2048
NH = 16
HD = 128
NA = 1024
REFF = 512
NKEEP = T - REFF
BN = 256
NW = 32
ROWS_PER = NKEEP // NW
UROWS = T // NW

_SCALE = HD ** -0.5

@functools.lru_cache(maxsize=None)
def _sc_mesh():
    return plsc.VectorSubcoreMesh(core_axis_name="c", subcore_axis_name="s")


def _m1_body(a_ref, b_ref, na_ref, nb_ref, bb_ref, rank_ref, cume_ref,
             bs_ref, abf_ref):
    j = pl.program_id(0)

    @pl.when(j == 0)
    def _():
        abf_ref[...] = (a_ref[...] / na_ref[...]).astype(jnp.bfloat16)
        bs_ref[...] = jnp.full((NA, 1), -jnp.inf, jnp.float32)
        bb_ref[...] = jnp.zeros((NA, 1), jnp.int32)

    bbf = (b_ref[...] / nb_ref[...]).astype(jnp.bfloat16)
    sim = lax.dot_general(
        abf_ref[...], bbf, (((1,), (1,)), ((), ())),
        preferred_element_type=jnp.float32,
    )
    bmax = jnp.max(sim, axis=1, keepdims=True)
    ids = lax.broadcasted_iota(jnp.int32, (NA, BN), 1) + j * BN
    bloc = jnp.min(jnp.where(sim == bmax, ids, NA), axis=1, keepdims=True)
    upd = bmax > bs_ref[...]
    bb_ref[...] = jnp.where(upd, bloc, bb_ref[...])
    bs_ref[...] = jnp.maximum(bmax, bs_ref[...])

    @pl.when(j == NA // BN - 1)
    def _():
        bs = bs_ref[...]
        rid = lax.broadcasted_iota(jnp.int32, (NA, NA), 0)
        cid = lax.broadcasted_iota(jnp.int32, (NA, NA), 1)
        ident = (rid == cid).astype(jnp.float32)
        nn = (((1,), (0,)), ((), ()))
        bs_row = lax.dot_general(
            jnp.ones((1, NA), jnp.float32), ident * bs, nn,
            preferred_element_type=jnp.float32,
            precision=lax.Precision.HIGHEST,
        )
        b_j = jnp.broadcast_to(bs_row, (NA, NA))
        b_i = jnp.broadcast_to(bs, (NA, NA))
        grt = (b_j > b_i) | ((b_j == b_i) & (cid < rid))
        rank = jnp.sum(grt.astype(jnp.float32), axis=1, keepdims=True)
        keep_e = jnp.where(rank >= float(REFF), 1.0, 0.0)
        tri = (cid <= rid).astype(jnp.float32)
        cume = lax.dot_general(
            tri, keep_e, nn,
            preferred_element_type=jnp.float32,
            precision=lax.Precision.HIGHEST,
        )
        rank_ref[...] = rank.astype(jnp.int32)
        cume_ref[...] = cume.astype(jnp.int32)


def _match_call(xr, na, nb):
    return pl.pallas_call(
        _m1_body,
        grid=(NA // BN,),
        in_specs=[
            pl.BlockSpec((NA, D), lambda j: (0, 0)),
            pl.BlockSpec((BN, D), lambda j: (j, 1)),
            pl.BlockSpec((NA, 1), lambda j: (0, 0)),
            pl.BlockSpec((BN, 1), lambda j: (j, 0)),
        ],
        out_specs=[
            pl.BlockSpec((NA, 1), lambda j: (0, 0)),
            pl.BlockSpec((NA, 1), lambda j: (0, 0)),
            pl.BlockSpec((NA, 1), lambda j: (0, 0)),
        ],
        out_shape=[
            jax.ShapeDtypeStruct((NA, 1), jnp.int32),
            jax.ShapeDtypeStruct((NA, 1), jnp.int32),
            jax.ShapeDtypeStruct((NA, 1), jnp.int32),
        ],
        scratch_shapes=[
            pltpu.VMEM((NA, 1), jnp.float32),
            pltpu.VMEM((NA, D), jnp.bfloat16),
        ],
    )(xr, xr, na, nb)


def _sc_route_body(x_hbm, rank_hbm, bb_hbm, cume_hbm, xm_hbm, lookup_hbm,
                   rank_v, bb_v, cume_v, src_v, dst_v, kidx_v, lk_v, rows_v,
                   sem):
    wid = lax.axis_index("s") * 2 + lax.axis_index("c")
    lo = wid * ROWS_PER

    pltpu.sync_copy(rank_hbm, rank_v)
    pltpu.sync_copy(bb_hbm, bb_v)
    pltpu.sync_copy(cume_hbm, cume_v)

    def body1(it, carry):
        i16 = jnp.arange(16, dtype=jnp.int32) + it * 16
        r16 = rank_v[pl.ds(it * 16, 16)]
        b16 = bb_v[pl.ds(it * 16, 16)]
        m = r16 < REFF
        plsc.store_scatter(src_v, [r16], 2 * i16, mask=m)
        plsc.store_scatter(dst_v, [r16], 2 * b16 + 1, mask=m)
        return carry

    lax.fori_loop(0, NA // 16, body1, 0)

    def body2(it, carry):
        t16 = jnp.arange(16, dtype=jnp.int32) + it * 16
        i16 = t16 >> 1
        odd = t16 & 1
        rk = plsc.load_gather(rank_v, [i16])
        ce = plsc.load_gather(cume_v, [i16])
        keep = (odd == 1) | (rk >= REFF)
        c = i16 + ce - 1 + odd
        plsc.store_scatter(kidx_v, [c], t16, mask=keep)
        lk_v[pl.ds(it * 16, 16)] = c
        return carry

    lax.fori_loop(0, T // 16, body2, 0)

    def body3(it, carry):
        s16 = src_v[pl.ds(it * 16, 16)]
        d16 = dst_v[pl.ds(it * 16, 16)]
        di = d16 >> 1
        ce = plsc.load_gather(cume_v, [di])
        plsc.store_scatter(lk_v, [s16], di + ce)
        return carry

    lax.fori_loop(0, REFF // 16, body3, 0)

    pltpu.async_copy(x_hbm.at[kidx_v.at[pl.ds(lo, ROWS_PER)]], rows_v, sem).wait()

    def body4(it, carry):
        r16 = jnp.arange(16, dtype=jnp.int32) + it * 16
        i16 = r16 >> 1
        odd = r16 & 1
        rk = plsc.load_gather(rank_v, [i16])
        ce = plsc.load_gather(cume_v, [i16])
        keep = (odd == 1) | (rk >= REFF)
        c = i16 + ce - 1 + odd
        m = keep & (c >= lo) & (c < lo + ROWS_PER)
        loc = c - lo
        s16 = src_v[pl.ds(it * 16, 16)]
        d16 = dst_v[pl.ds(it * 16, 16)]
        xs = plsc.load_gather(rows_v, [loc, s16], mask=m)
        xd = plsc.load_gather(rows_v, [loc, d16], mask=m)
        plsc.store_scatter(rows_v, [loc, d16], (xs + xd) * 0.5, mask=m)
        return carry

    lax.fori_loop(0, REFF // 16, body4, 0)

    pltpu.sync_copy(rows_v, xm_hbm.at[pl.ds(lo, ROWS_PER)])

    @pl.when(wid == 0)
    def _():
        pltpu.sync_copy(lk_v, lookup_hbm)


@functools.lru_cache(maxsize=None)
def _sc_route():
    return pl.kernel(
        _sc_route_body,
        out_type=(
            jax.ShapeDtypeStruct((NKEEP, D), jnp.float32),
            jax.ShapeDtypeStruct((T,), jnp.int32),
        ),
        mesh=_sc_mesh(),
        compiler_params=pltpu.CompilerParams(needs_layout_passes=False),
        scratch_types=[
            pltpu.VMEM((NA,), jnp.int32),
            pltpu.VMEM((NA,), jnp.int32),
            pltpu.VMEM((NA,), jnp.int32),
            pltpu.VMEM((REFF,), jnp.int32),
            pltpu.VMEM((REFF,), jnp.int32),
            pltpu.VMEM((NKEEP,), jnp.int32),
            pltpu.VMEM((T,), jnp.int32),
            pltpu.VMEM((ROWS_PER, D), jnp.float32),
            pltpu.SemaphoreType.DMA,
        ],
    )


def _sc_route_call(x2, rank, bb, cume):
    return _sc_route()(x2, rank, bb, cume)


def _qkv_body(xm_ref, w_ref, out_ref, xbf_ref):
    @pl.when(pl.program_id(0) == 0)
    def _():
        xbf_ref[...] = xm_ref[...].astype(jnp.bfloat16)

    w = w_ref[...].astype(jnp.bfloat16)
    out_ref[...] = lax.dot_general(
        xbf_ref[...], w, (((1,), (1,)), ((), ())),
        preferred_element_type=jnp.float32,
    ).astype(jnp.bfloat16)


def _qkv_call(xm, w_qkv):
    nblk = (3 * D) // 512
    return pl.pallas_call(
        _qkv_body,
        grid=(nblk,),
        in_specs=[
            pl.BlockSpec((NKEEP, D), lambda j: (0, 0)),
            pl.BlockSpec((512, D), lambda j: (j, 0)),
        ],
        out_specs=pl.BlockSpec((NKEEP, 512), lambda j: (0, j)),
        out_shape=jax.ShapeDtypeStruct((NKEEP, 3 * D), jnp.bfloat16),
        scratch_shapes=[pltpu.VMEM((NKEEP, D), jnp.bfloat16)],
    )(xm, w_qkv)


def _attn_body(q_ref, k_ref, v_ref, o_ref):
    nt = (((1,), (1,)), ((), ()))
    nn = (((1,), (0,)), ((), ()))
    logits = lax.dot_general(q_ref[...], k_ref[...], nt,
                             preferred_element_type=jnp.float32)
    p = jnp.exp(logits * _SCALE).astype(jnp.bfloat16)
    s = jnp.sum(p, axis=1, keepdims=True, dtype=jnp.float32)
    o = lax.dot_general(p, v_ref[...], nn, preferred_element_type=jnp.float32)
    o_ref[...] = (o / s).astype(jnp.bfloat16)


def _attn_call(qkv):
    return pl.pallas_call(
        _attn_body,
        grid=(NH,),
        in_specs=[
            pl.BlockSpec((NKEEP, HD), lambda h: (0, h)),
            pl.BlockSpec((NKEEP, HD), lambda h: (0, NH + h)),
            pl.BlockSpec((NKEEP, HD), lambda h: (0, 2 * NH + h)),
        ],
        out_specs=pl.BlockSpec((NKEEP, HD), lambda h: (0, h)),
        out_shape=jax.ShapeDtypeStruct((NKEEP, D), jnp.bfloat16),
    )(qkv, qkv, qkv)


def _projmerge_body(lk_ref, o_ref, w_ref, out_ref, ou_ref):
    nt = (((1,), (1,)), ((), ()))
    nn = (((1,), (0,)), ((), ()))

    @pl.when(pl.program_id(0) == 0)
    def _():
        cid = lax.broadcasted_iota(jnp.int32, (T, NKEEP), 1)
        pmat = (lk_ref[...] == cid).astype(jnp.bfloat16)
        ou_ref[...] = lax.dot_general(
            pmat, o_ref[...], nn,
            preferred_element_type=jnp.float32).astype(jnp.bfloat16)

    out_ref[...] = lax.dot_general(
        ou_ref[...], w_ref[...].astype(jnp.bfloat16), nt,
        preferred_element_type=jnp.float32)


def _projmerge_call(lookup, o_all, w_out):
    return pl.pallas_call(
        _projmerge_body,
        grid=(D // 512,),
        in_specs=[
            pl.BlockSpec((T, 1), lambda j: (0, 0)),
            pl.BlockSpec((NKEEP, D), lambda j: (0, 0)),
            pl.BlockSpec((512, D), lambda j: (j, 0)),
        ],
        out_specs=pl.BlockSpec((T, 512), lambda j: (0, j)),
        out_shape=jax.ShapeDtypeStruct((T, D), jnp.float32),
        scratch_shapes=[pltpu.VMEM((T, D), jnp.bfloat16)],
    )(lookup, o_all, w_out)


def _sc_unmerge_body(outr_hbm, lookup_hbm, out_hbm, lk_v, buf0, buf1,
                     sem0, sem1):
    wid = lax.axis_index("s") * 2 + lax.axis_index("c")
    lo = wid * UROWS
    pltpu.sync_copy(lookup_hbm.at[pl.ds(lo, UROWS)], lk_v)

    def start(ch, buf, sem):
        return pltpu.async_copy(outr_hbm.at[lk_v.at[pl.ds(ch * 16, 16)]], buf, sem)

    def drain(cp, ch, buf):
        cp.wait()
        pltpu.sync_copy(buf, out_hbm.at[pl.ds(lo + ch * 16, 16)])

    c0 = start(0, buf0, sem0)
    c1 = start(1, buf1, sem1)
    drain(c0, 0, buf0)
    c2 = start(2, buf0, sem0)
    drain(c1, 1, buf1)
    c3 = start(3, buf1, sem1)
    drain(c2, 2, buf0)
    drain(c3, 3, buf1)


@functools.lru_cache(maxsize=None)
def _sc_unmerge():
    return pl.kernel(
        _sc_unmerge_body,
        out_type=jax.ShapeDtypeStruct((T, D), jnp.float32),
        mesh=_sc_mesh(),
        compiler_params=pltpu.CompilerParams(needs_layout_passes=False),
        scratch_types=[
            pltpu.VMEM((UROWS,), jnp.int32),
            pltpu.VMEM((16, D), jnp.float32),
            pltpu.VMEM((16, D), jnp.float32),
            pltpu.SemaphoreType.DMA,
            pltpu.SemaphoreType.DMA,
        ],
    )


def _sc_unmerge_call(outr, lookup):
    return _sc_unmerge()(outr, lookup)


def kernel(x, W_qkv, W_out):
    x2 = x[0]
    xr = x2.reshape(NA, 2 * D)
    nrm = jnp.maximum(jnp.sqrt(jnp.sum(x2 * x2, axis=-1)), 1e-12)
    na = nrm[0::2].reshape(NA, 1)
    nb = nrm[1::2].reshape(NA, 1)
    bb, rank, cume = _match_call(xr, na, nb)
    xm, lookup = _sc_route_call(
        x2, rank.reshape(NA), bb.reshape(NA), cume.reshape(NA))
    qkv = _qkv_call(xm, W_qkv)
    o_all = _attn_call(qkv)
    out = _projmerge_call(lookup.reshape(T, 1), o_all, W_out)
    return out.reshape(1, T, D)


# speedup vs baseline: 1.6843x; 1.0040x over previous
import functools

import jax
import jax.numpy as jnp
from jax import lax
from jax.experimental import pallas as pl
from jax.experimental.pallas import tpu as pltpu
from jax.experimental.pallas import tpu_sc as plsc

T = 2048
D = 2048
NH = 16
HD = 128
NA = 1024
REFF = 512
NKEEP = T - REFF
BN = 256
NW = 32
ROWS_PER = NKEEP // NW
UROWS = T // NW

_SCALE = HD ** -0.5

@functools.lru_cache(maxsize=None)
def _sc_mesh():
    return plsc.VectorSubcoreMesh(core_axis_name="c", subcore_axis_name="s")


def _m1_body(a_ref, b_ref, na_ref, nb_ref, bb_ref, rank_ref, cume_ref,
             bs_ref, abf_ref):
    j = pl.program_id(0)

    @pl.when(j == 0)
    def _():
        abf_ref[...] = (a_ref[...] / na_ref[...]).astype(jnp.bfloat16)
        bs_ref[...] = jnp.full((NA, 1), -jnp.inf, jnp.float32)
        bb_ref[...] = jnp.zeros((NA, 1), jnp.int32)

    bbf = (b_ref[...] / nb_ref[...]).astype(jnp.bfloat16)
    sim = lax.dot_general(
        abf_ref[...], bbf, (((1,), (1,)), ((), ())),
        preferred_element_type=jnp.float32,
    )
    bmax = jnp.max(sim, axis=1, keepdims=True)
    ids = lax.broadcasted_iota(jnp.int32, (NA, BN), 1) + j * BN
    bloc = jnp.min(jnp.where(sim == bmax, ids, NA), axis=1, keepdims=True)
    upd = bmax > bs_ref[...]
    bb_ref[...] = jnp.where(upd, bloc, bb_ref[...])
    bs_ref[...] = jnp.maximum(bmax, bs_ref[...])

    @pl.when(j == NA // BN - 1)
    def _():
        bs = bs_ref[...]
        rid = lax.broadcasted_iota(jnp.int32, (NA, NA), 0)
        cid = lax.broadcasted_iota(jnp.int32, (NA, NA), 1)
        ident = (rid == cid).astype(jnp.float32)
        nn = (((1,), (0,)), ((), ()))
        bs_row = lax.dot_general(
            jnp.ones((1, NA), jnp.float32), ident * bs, nn,
            preferred_element_type=jnp.float32,
            precision=lax.Precision.HIGHEST,
        )
        b_j = jnp.broadcast_to(bs_row, (NA, NA))
        b_i = jnp.broadcast_to(bs, (NA, NA))
        grt = (b_j > b_i) | ((b_j == b_i) & (cid < rid))
        rank = jnp.sum(grt.astype(jnp.float32), axis=1, keepdims=True)
        keep_e = jnp.where(rank >= float(REFF), 1.0, 0.0)
        tri = (cid <= rid).astype(jnp.float32)
        cume = lax.dot_general(
            tri, keep_e, nn,
            preferred_element_type=jnp.float32,
            precision=lax.Precision.HIGHEST,
        )
        rank_ref[...] = rank.astype(jnp.int32)
        cume_ref[...] = cume.astype(jnp.int32)


def _match_call(xr, na, nb):
    return pl.pallas_call(
        _m1_body,
        grid=(NA // BN,),
        in_specs=[
            pl.BlockSpec((NA, D), lambda j: (0, 0)),
            pl.BlockSpec((BN, D), lambda j: (j, 1)),
            pl.BlockSpec((NA, 1), lambda j: (0, 0)),
            pl.BlockSpec((BN, 1), lambda j: (j, 0)),
        ],
        out_specs=[
            pl.BlockSpec((NA, 1), lambda j: (0, 0)),
            pl.BlockSpec((NA, 1), lambda j: (0, 0)),
            pl.BlockSpec((NA, 1), lambda j: (0, 0)),
        ],
        out_shape=[
            jax.ShapeDtypeStruct((NA, 1), jnp.int32),
            jax.ShapeDtypeStruct((NA, 1), jnp.int32),
            jax.ShapeDtypeStruct((NA, 1), jnp.int32),
        ],
        scratch_shapes=[
            pltpu.VMEM((NA, 1), jnp.float32),
            pltpu.VMEM((NA, D), jnp.bfloat16),
        ],
    )(xr, xr, na, nb)


def _sc_route_body(x_hbm, rank_hbm, bb_hbm, cume_hbm, xm_hbm, lookup_hbm,
                   rank_v, bb_v, cume_v, src_v, dst_v, kidx_v, lk_v, rows_v,
                   sem):
    wid = lax.axis_index("s") * 2 + lax.axis_index("c")
    lo = wid * ROWS_PER

    pltpu.sync_copy(rank_hbm, rank_v)
    pltpu.sync_copy(bb_hbm, bb_v)
    pltpu.sync_copy(cume_hbm, cume_v)

    def body1(it, carry):
        i16 = jnp.arange(16, dtype=jnp.int32) + it * 16
        r16 = rank_v[pl.ds(it * 16, 16)]
        b16 = bb_v[pl.ds(it * 16, 16)]
        m = r16 < REFF
        plsc.store_scatter(src_v, [r16], 2 * i16, mask=m)
        plsc.store_scatter(dst_v, [r16], 2 * b16 + 1, mask=m)
        return carry

    lax.fori_loop(0, NA // 16, body1, 0)

    def body2(it, carry):
        t16 = jnp.arange(16, dtype=jnp.int32) + it * 16
        i16 = t16 >> 1
        odd = t16 & 1
        rk = plsc.load_gather(rank_v, [i16])
        ce = plsc.load_gather(cume_v, [i16])
        keep = (odd == 1) | (rk >= REFF)
        c = i16 + ce - 1 + odd
        plsc.store_scatter(kidx_v, [c], t16, mask=keep)
        lk_v[pl.ds(it * 16, 16)] = c
        return carry

    lax.fori_loop(0, T // 16, body2, 0)

    def body3(it, carry):
        s16 = src_v[pl.ds(it * 16, 16)]
        d16 = dst_v[pl.ds(it * 16, 16)]
        di = d16 >> 1
        ce = plsc.load_gather(cume_v, [di])
        plsc.store_scatter(lk_v, [s16], di + ce)
        return carry

    lax.fori_loop(0, REFF // 16, body3, 0)

    pltpu.async_copy(x_hbm.at[kidx_v.at[pl.ds(lo, ROWS_PER)]], rows_v, sem).wait()

    def body4(it, carry):
        r16 = jnp.arange(16, dtype=jnp.int32) + it * 16
        i16 = r16 >> 1
        odd = r16 & 1
        rk = plsc.load_gather(rank_v, [i16])
        ce = plsc.load_gather(cume_v, [i16])
        keep = (odd == 1) | (rk >= REFF)
        c = i16 + ce - 1 + odd
        m = keep & (c >= lo) & (c < lo + ROWS_PER)
        loc = c - lo
        s16 = src_v[pl.ds(it * 16, 16)]
        d16 = dst_v[pl.ds(it * 16, 16)]
        xs = plsc.load_gather(rows_v, [loc, s16], mask=m)
        xd = plsc.load_gather(rows_v, [loc, d16], mask=m)
        plsc.store_scatter(rows_v, [loc, d16], (xs + xd) * 0.5, mask=m)
        return carry

    lax.fori_loop(0, REFF // 16, body4, 0)

    pltpu.sync_copy(rows_v, xm_hbm.at[pl.ds(lo, ROWS_PER)])

    @pl.when(wid == 0)
    def _():
        pltpu.sync_copy(lk_v, lookup_hbm)


@functools.lru_cache(maxsize=None)
def _sc_route():
    return pl.kernel(
        _sc_route_body,
        out_type=(
            jax.ShapeDtypeStruct((NKEEP, D), jnp.float32),
            jax.ShapeDtypeStruct((T,), jnp.int32),
        ),
        mesh=_sc_mesh(),
        compiler_params=pltpu.CompilerParams(needs_layout_passes=False),
        scratch_types=[
            pltpu.VMEM((NA,), jnp.int32),
            pltpu.VMEM((NA,), jnp.int32),
            pltpu.VMEM((NA,), jnp.int32),
            pltpu.VMEM((REFF,), jnp.int32),
            pltpu.VMEM((REFF,), jnp.int32),
            pltpu.VMEM((NKEEP,), jnp.int32),
            pltpu.VMEM((T,), jnp.int32),
            pltpu.VMEM((ROWS_PER, D), jnp.float32),
            pltpu.SemaphoreType.DMA,
        ],
    )


def _sc_route_call(x2, rank, bb, cume):
    return _sc_route()(x2, rank, bb, cume)


def _qkv_body(xm_ref, w_ref, out_ref, xbf_ref):
    @pl.when(pl.program_id(0) == 0)
    def _():
        xbf_ref[...] = xm_ref[...].astype(jnp.bfloat16)

    w = w_ref[...].astype(jnp.bfloat16)
    out_ref[...] = lax.dot_general(
        xbf_ref[...], w, (((1,), (1,)), ((), ())),
        preferred_element_type=jnp.float32,
    ).astype(jnp.bfloat16)


def _qkv_call(xm, w_qkv):
    blk = 1024
    return pl.pallas_call(
        _qkv_body,
        grid=((3 * D) // blk,),
        in_specs=[
            pl.BlockSpec((NKEEP, D), lambda j: (0, 0)),
            pl.BlockSpec((blk, D), lambda j: (j, 0)),
        ],
        out_specs=pl.BlockSpec((NKEEP, blk), lambda j: (0, j)),
        out_shape=jax.ShapeDtypeStruct((NKEEP, 3 * D), jnp.bfloat16),
        scratch_shapes=[pltpu.VMEM((NKEEP, D), jnp.bfloat16)],
    )(xm, w_qkv)


def _attn_body(q_ref, k_ref, v_ref, o_ref):
    nt = (((1,), (1,)), ((), ()))
    nn = (((1,), (0,)), ((), ()))
    logits = lax.dot_general(q_ref[...], k_ref[...], nt,
                             preferred_element_type=jnp.float32)
    p = jnp.exp(logits * _SCALE).astype(jnp.bfloat16)
    s = jnp.sum(p, axis=1, keepdims=True, dtype=jnp.float32)
    o = lax.dot_general(p, v_ref[...], nn, preferred_element_type=jnp.float32)
    o_ref[...] = (o / s).astype(jnp.bfloat16)


def _attn_call(qkv):
    return pl.pallas_call(
        _attn_body,
        grid=(NH,),
        in_specs=[
            pl.BlockSpec((NKEEP, HD), lambda h: (0, h)),
            pl.BlockSpec((NKEEP, HD), lambda h: (0, NH + h)),
            pl.BlockSpec((NKEEP, HD), lambda h: (0, 2 * NH + h)),
        ],
        out_specs=pl.BlockSpec((NKEEP, HD), lambda h: (0, h)),
        out_shape=jax.ShapeDtypeStruct((NKEEP, D), jnp.bfloat16),
    )(qkv, qkv, qkv)


def _projmerge_body(lk_ref, o_ref, w_ref, out_ref, ou_ref):
    nt = (((1,), (1,)), ((), ()))
    nn = (((1,), (0,)), ((), ()))

    @pl.when(pl.program_id(0) == 0)
    def _():
        cid = lax.broadcasted_iota(jnp.int32, (T, NKEEP), 1)
        pmat = (lk_ref[...] == cid).astype(jnp.bfloat16)
        ou_ref[...] = lax.dot_general(
            pmat, o_ref[...], nn,
            preferred_element_type=jnp.float32).astype(jnp.bfloat16)

    out_ref[...] = lax.dot_general(
        ou_ref[...], w_ref[...].astype(jnp.bfloat16), nt,
        preferred_element_type=jnp.float32)


def _projmerge_call(lookup, o_all, w_out):
    return pl.pallas_call(
        _projmerge_body,
        grid=(D // 512,),
        in_specs=[
            pl.BlockSpec((T, 1), lambda j: (0, 0)),
            pl.BlockSpec((NKEEP, D), lambda j: (0, 0)),
            pl.BlockSpec((512, D), lambda j: (j, 0)),
        ],
        out_specs=pl.BlockSpec((T, 512), lambda j: (0, j)),
        out_shape=jax.ShapeDtypeStruct((T, D), jnp.float32),
        scratch_shapes=[pltpu.VMEM((T, D), jnp.bfloat16)],
    )(lookup, o_all, w_out)


def _sc_unmerge_body(outr_hbm, lookup_hbm, out_hbm, lk_v, buf0, buf1,
                     sem0, sem1):
    wid = lax.axis_index("s") * 2 + lax.axis_index("c")
    lo = wid * UROWS
    pltpu.sync_copy(lookup_hbm.at[pl.ds(lo, UROWS)], lk_v)

    def start(ch, buf, sem):
        return pltpu.async_copy(outr_hbm.at[lk_v.at[pl.ds(ch * 16, 16)]], buf, sem)

    def drain(cp, ch, buf):
        cp.wait()
        pltpu.sync_copy(buf, out_hbm.at[pl.ds(lo + ch * 16, 16)])

    c0 = start(0, buf0, sem0)
    c1 = start(1, buf1, sem1)
    drain(c0, 0, buf0)
    c2 = start(2, buf0, sem0)
    drain(c1, 1, buf1)
    c3 = start(3, buf1, sem1)
    drain(c2, 2, buf0)
    drain(c3, 3, buf1)


@functools.lru_cache(maxsize=None)
def _sc_unmerge():
    return pl.kernel(
        _sc_unmerge_body,
        out_type=jax.ShapeDtypeStruct((T, D), jnp.float32),
        mesh=_sc_mesh(),
        compiler_params=pltpu.CompilerParams(needs_layout_passes=False),
        scratch_types=[
            pltpu.VMEM((UROWS,), jnp.int32),
            pltpu.VMEM((16, D), jnp.float32),
            pltpu.VMEM((16, D), jnp.float32),
            pltpu.SemaphoreType.DMA,
            pltpu.SemaphoreType.DMA,
        ],
    )


def _sc_unmerge_call(outr, lookup):
    return _sc_unmerge()(outr, lookup)


def kernel(x, W_qkv, W_out):
    x2 = x[0]
    xr = x2.reshape(NA, 2 * D)
    nrm = jnp.maximum(jnp.sqrt(jnp.sum(x2 * x2, axis=-1)), 1e-12)
    na = nrm[0::2].reshape(NA, 1)
    nb = nrm[1::2].reshape(NA, 1)
    bb, rank, cume = _match_call(xr, na, nb)
    xm, lookup = _sc_route_call(
        x2, rank.reshape(NA), bb.reshape(NA), cume.reshape(NA))
    qkv = _qkv_call(xm, W_qkv)
    o_all = _attn_call(qkv)
    out = _projmerge_call(lookup.reshape(T, 1), o_all, W_out)
    return out.reshape(1, T, D)
